# baseline (speedup 1.0000x reference)
import functools

import jax
import jax.numpy as jnp
from jax import lax
from jax.experimental import pallas as pl
from jax.experimental.pallas import tpu as pltpu
from jax.experimental.pallas import tpu_sc as plsc

EMBED_DIM = 32
BBLK = 128
NUM_WORKERS = 32


def _sc_lookup_t(sents_t, table):
    seq_len, batch = sents_t.shape
    n_units = seq_len

    mesh = plsc.VectorSubcoreMesh(core_axis_name="c", subcore_axis_name="s")

    @functools.partial(
        pl.kernel,
        mesh=mesh,
        out_type=jax.ShapeDtypeStruct((seq_len, EMBED_DIM, batch), jnp.float32),
        scratch_types=[
            pltpu.VMEM((seq_len, BBLK), jnp.int32),
            pltpu.VMEM((2, BBLK, EMBED_DIM), jnp.float32),
            pltpu.VMEM((2, EMBED_DIM, BBLK), jnp.float32),
            pltpu.SemaphoreType.DMA,
        ],
        compiler_params=pltpu.CompilerParams(
            use_tc_tiling_on_sc=False, needs_layout_passes=False
        ),
    )
    def k(table_hbm, idx_hbm, out_hbm, idx_v, rows_v, outb_v, gsem):
        wid = lax.axis_index("s") * 2 + lax.axis_index("c")
        b0 = wid * BBLK

        pltpu.sync_copy(idx_hbm.at[:, pl.ds(b0, BBLK)], idx_v)

        def start_gather(s, b):
            pltpu.async_copy(table_hbm.at[idx_v.at[s]], rows_v.at[b], gsem)

        def wait_gather(b):
            pltpu.make_async_copy(
                table_hbm.at[idx_v.at[0]], rows_v.at[b], gsem
            ).wait()

        lanes = lax.iota(jnp.int32, 16)
        row_sel = [lanes + 16 * g for g in range(BBLK // 16)]
        zeros = lanes - lanes
        col_sel = [zeros + d for d in range(EMBED_DIM)]

        start_gather(0, 0)

        def body(s, carry):
            for b in range(2):
                s_cur = 2 * s + b
                wait_gather(b)

                @pl.when(s_cur + 1 < n_units)
                def _():
                    start_gather(s_cur + 1, 1 - b)

                for d in range(EMBED_DIM):
                    for g in range(BBLK // 16):
                        vals = plsc.load_gather(
                            rows_v.at[b], [row_sel[g], col_sel[d]]
                        )
                        outb_v[b, d, pl.ds(16 * g, 16)] = vals
                pltpu.sync_copy(
                    outb_v.at[b],
                    out_hbm.at[s_cur, :, pl.ds(b0, BBLK)],
                )
            return carry

        lax.fori_loop(0, n_units // 2, body, 0)

    return k(table, sents_t)


def kernel(sents, table):
    if sents.ndim < 2:
        sents = sents[None, :]
    batch, seq_len = sents.shape
    out_t = _sc_lookup_t(sents.T.astype(jnp.int32), table)
    return out_t.transpose(2, 0, 1)


# speedup vs baseline: 1.3265x; 1.3265x over previous
import functools

import jax
import jax.numpy as jnp
from jax import lax
from jax.experimental import pallas as pl
from jax.experimental.pallas import tpu as pltpu
from jax.experimental.pallas import tpu_sc as plsc

EMBED_DIM = 32
BBLK = 128
NUM_WORKERS = 32


def _sc_lookup_t(sents_t, table):
    seq_len, batch = sents_t.shape
    n_units = seq_len

    mesh = plsc.VectorSubcoreMesh(core_axis_name="c", subcore_axis_name="s")

    @functools.partial(
        pl.kernel,
        mesh=mesh,
        out_type=jax.ShapeDtypeStruct((seq_len, EMBED_DIM, batch), jnp.float32),
        scratch_types=[
            pltpu.VMEM((seq_len, BBLK), jnp.int32),
            pltpu.VMEM((2, BBLK, EMBED_DIM), jnp.float32),
            pltpu.VMEM((2, EMBED_DIM, BBLK), jnp.float32),
            pltpu.SemaphoreType.DMA,
        ],
        compiler_params=pltpu.CompilerParams(
            use_tc_tiling_on_sc=False, needs_layout_passes=False
        ),
    )
    def k(table_hbm, idx_hbm, out_hbm, idx_v, rows_v, outb_v, gsem):
        wid = lax.axis_index("s") * 2 + lax.axis_index("c")
        b0 = wid * BBLK

        pltpu.sync_copy(idx_hbm.at[:, pl.ds(b0, BBLK)], idx_v)

        def start_gather(s, b):
            pltpu.async_copy(table_hbm.at[idx_v.at[s]], rows_v.at[b], gsem)

        def wait_gather(b):
            pltpu.make_async_copy(
                table_hbm.at[idx_v.at[0]], rows_v.at[b], gsem
            ).wait()

        lanes = lax.iota(jnp.int32, 16)
        bsel = [lanes + 16 * g for g in range(BBLK // 16)]
        dsel = [(lanes + k) & (EMBED_DIM - 1) for k in range(EMBED_DIM)]

        start_gather(0, 0)

        def body(s, carry):
            for b in range(2):
                s_cur = 2 * s + b
                wait_gather(b)

                @pl.when(s_cur + 1 < n_units)
                def _():
                    start_gather(s_cur + 1, 1 - b)

                for k in range(EMBED_DIM):
                    for g in range(BBLK // 16):
                        vals = plsc.load_gather(
                            rows_v.at[b], [bsel[g], dsel[k]]
                        )
                        plsc.store_scatter(
                            outb_v.at[b], [dsel[k], bsel[g]], vals
                        )
                pltpu.sync_copy(
                    outb_v.at[b],
                    out_hbm.at[s_cur, :, pl.ds(b0, BBLK)],
                )
            return carry

        lax.fori_loop(0, n_units // 2, body, 0)

    return k(table, sents_t)


def kernel(sents, table):
    if sents.ndim < 2:
        sents = sents[None, :]
    batch, seq_len = sents.shape
    out_t = _sc_lookup_t(sents.T.astype(jnp.int32), table)
    return out_t.transpose(2, 0, 1)


# speedup vs baseline: 2.0255x; 1.5270x over previous
import functools

import jax
import jax.numpy as jnp
from jax import lax
from jax.experimental import pallas as pl
from jax.experimental.pallas import tpu as pltpu
from jax.experimental.pallas import tpu_sc as plsc

VOCAB_ROWS = 1000000
EMBED_DIM = 32
BBLK = 128
NUM_WORKERS = 32

REPACK_VBLK = 2048


def _tc_repack_kernel(tt_ref, out_ref):
    y = tt_ref[...].T
    out_ref[...] = jnp.concatenate([y[j::4, :] for j in range(4)], axis=1)


def _tc_repack(table_t):
    return pl.pallas_call(
        _tc_repack_kernel,
        grid=(pl.cdiv(VOCAB_ROWS, REPACK_VBLK),),
        in_specs=[
            pl.BlockSpec((EMBED_DIM, REPACK_VBLK), lambda g: (0, g)),
        ],
        out_specs=pl.BlockSpec(
            (REPACK_VBLK // 4, 4 * EMBED_DIM), lambda g: (g, 0)
        ),
        out_shape=jax.ShapeDtypeStruct(
            (VOCAB_ROWS // 4, 4 * EMBED_DIM), jnp.float32
        ),
    )(table_t)


def _tc_out_transpose_kernel(x_ref, out_ref):
    out_ref[...] = x_ref[...].T


def _tc_out_transpose(x, seq_len, batch):
    return pl.pallas_call(
        _tc_out_transpose_kernel,
        grid=(seq_len // 4,),
        in_specs=[pl.BlockSpec((batch, 4 * EMBED_DIM), lambda g: (g, 0))],
        out_specs=pl.BlockSpec((4 * EMBED_DIM, batch), lambda g: (g, 0)),
        out_shape=jax.ShapeDtypeStruct(
            (seq_len * EMBED_DIM, batch), jnp.float32
        ),
    )(x)


def _sc_gather(sents_t, table_rm):
    seq_len, batch = sents_t.shape

    mesh = plsc.VectorSubcoreMesh(core_axis_name="c", subcore_axis_name="s")

    @functools.partial(
        pl.kernel,
        mesh=mesh,
        out_type=jax.ShapeDtypeStruct(
            (seq_len * batch // 4, 4 * EMBED_DIM), jnp.float32
        ),
        scratch_types=[
            pltpu.VMEM((seq_len, BBLK), jnp.int32),
            pltpu.VMEM((4, BBLK, EMBED_DIM), jnp.float32),
            pltpu.SemaphoreType.DMA,
        ],
        compiler_params=pltpu.CompilerParams(use_tc_tiling_on_sc=False),
    )
    def k(table_hbm, idx_hbm, out_hbm, idx_v, rows_v, gsem):
        wid = lax.axis_index("s") * 2 + lax.axis_index("c")
        b0 = wid * BBLK

        pltpu.sync_copy(idx_hbm.at[:, pl.ds(b0, BBLK)], idx_v)

        def start_gather(s, b):
            pltpu.async_copy(table_hbm.at[idx_v.at[s]], rows_v.at[b], gsem)

        def wait_gather(b):
            pltpu.make_async_copy(
                table_hbm.at[idx_v.at[0]], rows_v.at[b], gsem
            ).wait()

        for b in range(4):
            start_gather(b, b)

        def body(g, carry):
            for b in range(4):
                s = g * 4 + b
                wait_gather(b)
                pltpu.sync_copy(
                    rows_v.at[b],
                    out_hbm.at[
                        pl.ds(g * batch + b0, BBLK),
                        pl.ds(b * EMBED_DIM, EMBED_DIM),
                    ],
                )

                @pl.when(s + 4 < seq_len)
                def _():
                    start_gather(s + 4, b)

            return carry

        lax.fori_loop(0, seq_len // 4, body, 0)

    return k(table_rm, sents_t)


def kernel(sents, table):
    if sents.ndim < 2:
        sents = sents[None, :]
    batch, seq_len = sents.shape

    x2 = _sc_gather(sents.T.astype(jnp.int32), table)
    out2d = _tc_out_transpose(x2, seq_len, batch)
    return out2d.reshape(seq_len, EMBED_DIM, batch).transpose(2, 0, 1)
